# baseline (speedup 1.0000x reference)
import functools

import jax
import jax.numpy as jnp
from jax import lax
from jax.experimental import pallas as pl
from jax.experimental.pallas import tpu as pltpu
from jax.experimental.pallas import tpu_sc as plsc

D = 768
L = 16
VECS = D // L
NC = 2
NS = 16
NW = NC * NS
B = 4 * 2048
BPW = B // NW
C = 64
NCHUNK = BPW // C
EPS = 1.1920928955078125e-07


def _rsqrt_vec(x):
    i = lax.bitcast_convert_type(x, jnp.int32)
    y = lax.bitcast_convert_type(jnp.int32(0x5F3759DF) - (i >> 1), jnp.float32)
    for _ in range(3):
        y = y * (1.5 - 0.5 * x * y * y)
    return y


_GATHER_DNUMS = lax.GatherDimensionNumbers(
    offset_dims=(), collapsed_slice_dims=(0,), start_index_map=(0,)
)


def _shuffle(v, idx):
    return lax.gather(
        v,
        idx[:, None],
        _GATHER_DNUMS,
        slice_sizes=(1,),
        mode=lax.GatherScatterMode.PROMISE_IN_BOUNDS,
    )


def _xlane_sum(v):
    lanes = lax.iota(jnp.int32, L)
    for k in (8, 4, 2, 1):
        v = v + _shuffle(v, lanes ^ k)
    return v


def _sc_body(tokens_hbm, table_hbm, out_hbm, idx_v, rows_v, sem):
    wid = lax.axis_index("s") * NC + lax.axis_index("c")
    pltpu.sync_copy(tokens_hbm.at[wid], idx_v)

    for j in range(NCHUNK):
        pltpu.async_copy(table_hbm.at[idx_v.at[j]], rows_v, sem).wait()

        def row_body(r, carry):
            acc = jnp.zeros((L,), jnp.float32)
            for k in range(VECS):
                v = rows_v[r, pl.ds(k * L, L)]
                acc = acc + v * v
            ms = _xlane_sum(acc) * (1.0 / D) + EPS
            s = _rsqrt_vec(ms)
            for k in range(VECS):
                rows_v[r, pl.ds(k * L, L)] = rows_v[r, pl.ds(k * L, L)] * s
            return carry

        lax.fori_loop(0, C, row_body, 0)
        pltpu.sync_copy(rows_v, out_hbm.at[pl.ds(wid * BPW + j * C, C)])


@jax.jit
def _sc_embed(tokens, table):
    mesh = plsc.VectorSubcoreMesh(
        core_axis_name="c", subcore_axis_name="s", num_cores=NC, num_subcores=NS
    )
    fn = pl.kernel(
        _sc_body,
        out_type=jax.ShapeDtypeStruct((B, D), jnp.float32),
        mesh=mesh,
        scratch_types=[
            pltpu.VMEM((NCHUNK, C), jnp.int32),
            pltpu.VMEM((C, D), jnp.float32),
            pltpu.SemaphoreType.DMA,
        ],
    )
    return fn(tokens, table)


def kernel(tokens, byte_tensor, byte_tensor_pulled, embed_tokens_weight):
    idx = tokens.reshape(NW, NCHUNK, C)
    out = _sc_embed(idx, embed_tokens_weight)
    return (out.reshape(tokens.shape + (D,)), None)


# speedup vs baseline: 1.1233x; 1.1233x over previous
import jax
import jax.numpy as jnp
from jax import lax
from jax.experimental import pallas as pl
from jax.experimental.pallas import tpu as pltpu
from jax.experimental.pallas import tpu_sc as plsc

D = 768
L = 16
VECS = D // L
NC = 2
NS = 16
NW = NC * NS
B = 4 * 2048
BPW = B // NW
C = 32
NCHUNK = BPW // C
NBUF = 4
EPS = 1.1920928955078125e-07

_GATHER_DNUMS = lax.GatherDimensionNumbers(
    offset_dims=(), collapsed_slice_dims=(0,), start_index_map=(0,)
)


def _shuffle(v, idx):
    return lax.gather(
        v,
        idx[:, None],
        _GATHER_DNUMS,
        slice_sizes=(1,),
        mode=lax.GatherScatterMode.PROMISE_IN_BOUNDS,
    )


def _xlane_sum(v):
    lanes = lax.iota(jnp.int32, L)
    for k in (8, 4, 2, 1):
        v = v + _shuffle(v, lanes ^ k)
    return v


def _rsqrt_vec(x):
    i = lax.bitcast_convert_type(x, jnp.int32)
    y = lax.bitcast_convert_type(jnp.int32(0x5F3759DF) - (i >> 1), jnp.float32)
    for _ in range(3):
        y = y * (1.5 - 0.5 * x * y * y)
    return y


def _normalize_chunk(buf):
    def row_body(r, carry):
        acc = jnp.zeros((L,), jnp.float32)
        for k in range(VECS):
            v = buf[r, pl.ds(k * L, L)]
            acc = acc + v * v
        ms = _xlane_sum(acc) * (1.0 / D) + EPS
        s = _rsqrt_vec(ms)
        for k in range(VECS):
            buf[r, pl.ds(k * L, L)] = buf[r, pl.ds(k * L, L)] * s
        return carry

    lax.fori_loop(0, C, row_body, 0)


def _sc_body(tokens_hbm, table_hbm, out_hbm, idx_v, bufs, gsems, ssems):
    wid = lax.axis_index("s") * NC + lax.axis_index("c")
    pltpu.sync_copy(tokens_hbm.at[wid], idx_v)

    def gather(j):
        return pltpu.make_async_copy(
            table_hbm.at[idx_v.at[j]], bufs[j % NBUF], gsems[j % NBUF]
        )

    def store(j):
        return pltpu.make_async_copy(
            bufs[j % NBUF], out_hbm.at[pl.ds(wid * BPW + j * C, C)], ssems[j % NBUF]
        )

    for j in range(min(NBUF - 1, NCHUNK)):
        gather(j).start()

    for j in range(NCHUNK):
        gather(j).wait()
        _normalize_chunk(bufs[j % NBUF])
        store(j).start()
        nxt = j + NBUF - 1
        if nxt < NCHUNK:
            if nxt >= NBUF:
                store(nxt - NBUF).wait()
            gather(nxt).start()

    for j in range(max(0, NCHUNK - NBUF), NCHUNK):
        store(j).wait()


@jax.jit
def _sc_embed(tokens, table):
    mesh = plsc.VectorSubcoreMesh(
        core_axis_name="c", subcore_axis_name="s", num_cores=NC, num_subcores=NS
    )
    fn = pl.kernel(
        _sc_body,
        out_type=jax.ShapeDtypeStruct((B, D), jnp.float32),
        mesh=mesh,
        scratch_types=[
            pltpu.VMEM((NCHUNK, C), jnp.int32),
            [pltpu.VMEM((C, D), jnp.float32) for _ in range(NBUF)],
            [pltpu.SemaphoreType.DMA for _ in range(NBUF)],
            [pltpu.SemaphoreType.DMA for _ in range(NBUF)],
        ],
    )
    return fn(tokens, table)


def kernel(tokens, byte_tensor, byte_tensor_pulled, embed_tokens_weight):
    idx = tokens.reshape(NW, NCHUNK, C)
    out = _sc_embed(idx, embed_tokens_weight)
    return (out.reshape(tokens.shape + (D,)), None)
